# Initial kernel scaffold; baseline (speedup 1.0000x reference)
import functools

import jax
import jax.numpy as jnp
from jax import lax
from jax.experimental import pallas as pl
from jax.experimental.pallas import tpu as pltpu
from jax.experimental.pallas import tpu_sc as plsc

B, T, D, F, E = 1, 2048, 768, 3072, 8
PRIME = 5099
CAP = T // E
NTILES = 32
TPT = T // NTILES
VPT = TPT // 16
EB = E + 1
ROWS = EB * CAP
TRASH = E * CAP


def _shift_body(x_ref, ss_ref, mk_ref, mr_ref, xk_ref, xr_ref):
    x = x_ref[...]
    rolled = pltpu.roll(x, shift=1, axis=0)
    ridx = lax.broadcasted_iota(jnp.int32, (T, D), 0)
    prev = jnp.where(ridx == 0, ss_ref[...], rolled)
    dx = prev - x
    xk_ref[...] = x + dx * mk_ref[...]
    xr_ref[...] = x + dx * mr_ref[...]


def _shift(xf, ss, mk, mr):
    return pl.pallas_call(
        _shift_body,
        out_shape=(
            jax.ShapeDtypeStruct((T, D), jnp.float32),
            jax.ShapeDtypeStruct((T, D), jnp.float32),
        ),
    )(xf, ss, mk, mr)


_SC_MESH = plsc.VectorSubcoreMesh(core_axis_name="c", subcore_axis_name="s")


def _expert_of(tid):
    return (tid * PRIME) & (E - 1)


@functools.partial(
    pl.kernel,
    out_type=(
        jax.ShapeDtypeStruct((ROWS, D), jnp.float32),
        jax.ShapeDtypeStruct((ROWS, D), jnp.float32),
        jax.ShapeDtypeStruct((T,), jnp.int32),
    ),
    mesh=_SC_MESH,
    scratch_types=[
        pltpu.VMEM((T,), jnp.int32),
        pltpu.VMEM((TPT,), jnp.int32),
        pltpu.VMEM((TPT, D), jnp.float32),
        pltpu.VMEM((TPT, D), jnp.float32),
        pltpu.SemaphoreType.DMA,
        pltpu.SemaphoreType.DMA,
    ],
)
def _route_dispatch(tid_hbm, xk_hbm, xr_hbm, dk_hbm, dr_hbm, g_hbm,
                    tid_v, g_v, xk_v, xr_v, sem_k, sem_r):
    w = lax.axis_index("s") * 2 + lax.axis_index("c")
    base_tok = w * TPT
    pltpu.sync_copy(tid_hbm, tid_v)

    def hist_step(i, bases):
        ev = _expert_of(tid_v[pl.ds(i * 16, 16)])
        return tuple(bases[ei] + jnp.sum((ev == ei).astype(jnp.int32))
                     for ei in range(E))

    zero = jnp.int32(0)
    bases = lax.fori_loop(0, w * VPT, hist_step, (zero,) * E)
    bases = list(bases)

    for j in range(VPT):
        ev = _expert_of(tid_v[pl.ds(base_tok + j * 16, 16)])
        pos = jnp.zeros((16,), jnp.int32)
        for ei in range(E):
            mi = (ev == ei).astype(jnp.int32)
            r = jnp.cumsum(mi)
            pos = jnp.where(ev == ei, bases[ei] + r - 1, pos)
            bases[ei] = bases[ei] + jnp.sum(mi)
        g = jnp.where(pos < CAP, ev * CAP + pos, TRASH)
        g_v[pl.ds(j * 16, 16)] = g

    pltpu.sync_copy(g_v, g_hbm.at[pl.ds(base_tok, TPT)])
    pltpu.sync_copy(xk_hbm.at[pl.ds(base_tok, TPT)], xk_v)
    ck = pltpu.async_copy(xk_v, dk_hbm.at[g_v], sem_k)
    pltpu.sync_copy(xr_hbm.at[pl.ds(base_tok, TPT)], xr_v)
    cr = pltpu.async_copy(xr_v, dr_hbm.at[g_v], sem_r)
    ck.wait()
    cr.wait()


def _ffn_body(dk_ref, dr_ref, wk_ref, wv_ref, wr_ref, oe_ref):
    e = pl.program_id(0)

    @pl.when(e < E)
    def _():
        dk = dk_ref[...]
        h = lax.dot_general(dk, wk_ref[0], (((1,), (1,)), ((), ())),
                            preferred_element_type=jnp.float32)
        h = jnp.maximum(h, 0.0)
        h = h * h
        ev = lax.dot_general(h, wv_ref[0], (((1,), (1,)), ((), ())),
                             preferred_element_type=jnp.float32)
        er = lax.dot_general(dr_ref[...], wr_ref[0], (((1,), (1,)), ((), ())),
                             preferred_element_type=jnp.float32)
        oe_ref[...] = ev * jax.nn.sigmoid(er)

    @pl.when(e == E)
    def _():
        oe_ref[...] = jnp.zeros_like(oe_ref)


def _ffn(dk, dr, Wk, Wv, Wr):
    clamp = lambda e: (jnp.minimum(e, E - 1), 0)
    clamp3 = lambda e: (jnp.minimum(e, E - 1), 0, 0)
    return pl.pallas_call(
        _ffn_body,
        grid=(EB,),
        in_specs=[
            pl.BlockSpec((CAP, D), clamp),
            pl.BlockSpec((CAP, D), clamp),
            pl.BlockSpec((1, F, D), clamp3),
            pl.BlockSpec((1, D, F), clamp3),
            pl.BlockSpec((1, D, D), clamp3),
        ],
        out_specs=pl.BlockSpec((CAP, D), lambda e: (e, 0)),
        out_shape=jax.ShapeDtypeStruct((ROWS, D), jnp.float32),
    )(dk, dr, Wk, Wv, Wr)


@functools.partial(
    pl.kernel,
    out_type=jax.ShapeDtypeStruct((T, D), jnp.float32),
    mesh=_SC_MESH,
    scratch_types=[
        pltpu.VMEM((TPT,), jnp.int32),
        pltpu.VMEM((TPT, D), jnp.float32),
        pltpu.SemaphoreType.DMA,
    ],
)
def _combine(g_hbm, oe_hbm, out_hbm, g_v, rows_v, sem):
    w = lax.axis_index("s") * 2 + lax.axis_index("c")
    base_tok = w * TPT
    pltpu.sync_copy(g_hbm.at[pl.ds(base_tok, TPT)], g_v)
    pltpu.async_copy(oe_hbm.at[g_v], rows_v, sem).wait()
    pltpu.sync_copy(rows_v, out_hbm.at[pl.ds(base_tok, TPT)])


def kernel(x, token_ids, shift_state, time_maa_k, time_maa_r, Wk, Wv, Wr):
    xf = x.reshape(T, D)
    ss = shift_state.reshape(1, D)
    mk = time_maa_k.reshape(1, D)
    mr = time_maa_r.reshape(1, D)
    xk, xr = _shift(xf, ss, mk, mr)
    tid = token_ids.reshape(T).astype(jnp.int32)
    dk, dr, g = _route_dispatch(tid, xk, xr)
    oe = _ffn(dk, dr, Wk, Wv, Wr)
    out = _combine(g, oe)
    return out.reshape(B, T, D), x[:, -1]


# speedup vs baseline: 2.7319x; 2.7319x over previous
import functools

import jax
import jax.numpy as jnp
from jax import lax
from jax.experimental import pallas as pl
from jax.experimental.pallas import tpu as pltpu
from jax.experimental.pallas import tpu_sc as plsc

B, T, D, F, E = 1, 2048, 768, 3072, 8
PRIME = 5099
CAP = T // E
NTILES = 32
TPT = T // NTILES
EB = E + 1
ROWS = EB * CAP
TRASH = E * CAP


def _shift_body(x_ref, ss_ref, mk_ref, mr_ref, tid_ref,
                xk_ref, xr_ref, g_ref):
    x = x_ref[...]
    rolled = pltpu.roll(x, shift=1, axis=0)
    ridx = lax.broadcasted_iota(jnp.int32, (T, D), 0)
    prev = jnp.where(ridx == 0, ss_ref[...], rolled)
    dx = prev - x
    xk_ref[...] = x + dx * mk_ref[...]
    xr_ref[...] = x + dx * mr_ref[...]

    e2 = (tid_ref[...] * PRIME) & (E - 1)
    jr = lax.broadcasted_iota(jnp.int32, (TPT, TPT), 0)
    jc = lax.broadcasted_iota(jnp.int32, (TPT, TPT), 1)
    ltj = jnp.where(jr < jc, 1.0, 0.0)
    wr_ = lax.broadcasted_iota(jnp.int32, (NTILES, NTILES), 0)
    wc_ = lax.broadcasted_iota(jnp.int32, (NTILES, NTILES), 1)
    ltw = jnp.where(wr_ > wc_, 1.0, 0.0)
    onesr = jnp.full((TPT, NTILES), 1.0, jnp.float32)
    pos = jnp.zeros((NTILES, TPT), jnp.float32)
    for ei in range(E):
        m = jnp.where(e2 == ei, 1.0, 0.0)
        pre = lax.dot_general(m, ltj, (((1,), (0,)), ((), ())),
                              preferred_element_type=jnp.float32)
        rowtot = lax.dot_general(m, onesr, (((1,), (0,)), ((), ())),
                                 preferred_element_type=jnp.float32)
        rowbase = lax.dot_general(ltw, rowtot, (((1,), (0,)), ((), ())),
                                  preferred_element_type=jnp.float32)
        pos = pos + m * (pre + rowbase[:, 0:1])
    posi = pos.astype(jnp.int32)
    g_ref[...] = jnp.where(posi < CAP, e2 * CAP + posi, TRASH)


def _shift_and_route(xf, ss, mk, mr, tid2d):
    return pl.pallas_call(
        _shift_body,
        out_shape=(
            jax.ShapeDtypeStruct((T, D), jnp.float32),
            jax.ShapeDtypeStruct((T, D), jnp.float32),
            jax.ShapeDtypeStruct((NTILES, TPT), jnp.int32),
        ),
    )(xf, ss, mk, mr, tid2d)


def _dispatch_body(g_hbm, xk_hbm, xr_hbm, dk_hbm, dr_hbm,
                   g_v, xk_v, xr_v, sem_k, sem_r):
    w = lax.axis_index("s") * 2 + lax.axis_index("c")
    base_tok = w * TPT
    pltpu.sync_copy(g_hbm.at[pl.ds(base_tok, TPT)], g_v)
    pltpu.sync_copy(xk_hbm.at[pl.ds(base_tok, TPT)], xk_v)
    ck = pltpu.async_copy(xk_v, dk_hbm.at[g_v], sem_k)
    pltpu.sync_copy(xr_hbm.at[pl.ds(base_tok, TPT)], xr_v)
    cr = pltpu.async_copy(xr_v, dr_hbm.at[g_v], sem_r)
    ck.wait()
    cr.wait()


@functools.lru_cache
def _dispatch():
    mesh = plsc.VectorSubcoreMesh(core_axis_name="c", subcore_axis_name="s")
    return pl.kernel(
        _dispatch_body,
        out_type=(
            jax.ShapeDtypeStruct((ROWS, D), jnp.float32),
            jax.ShapeDtypeStruct((ROWS, D), jnp.float32),
        ),
        mesh=mesh,
        scratch_types=[
            pltpu.VMEM((TPT,), jnp.int32),
            pltpu.VMEM((TPT, D), jnp.float32),
            pltpu.VMEM((TPT, D), jnp.float32),
            pltpu.SemaphoreType.DMA,
            pltpu.SemaphoreType.DMA,
        ],
    )


def _ffn_body(dk_ref, dr_ref, wk_ref, wv_ref, wr_ref, oe_ref):
    e = pl.program_id(0)

    @pl.when(e < E)
    def _():
        dk = dk_ref[...]
        h = lax.dot_general(dk, wk_ref[0], (((1,), (1,)), ((), ())),
                            preferred_element_type=jnp.float32)
        h = jnp.maximum(h, 0.0)
        h = h * h
        ev = lax.dot_general(h, wv_ref[0], (((1,), (1,)), ((), ())),
                             preferred_element_type=jnp.float32)
        er = lax.dot_general(dr_ref[...], wr_ref[0], (((1,), (1,)), ((), ())),
                             preferred_element_type=jnp.float32)
        oe_ref[...] = ev * jax.nn.sigmoid(er)

    @pl.when(e == E)
    def _():
        oe_ref[...] = jnp.zeros_like(oe_ref)


def _ffn(dk, dr, Wk, Wv, Wr):
    clamp = lambda e: (jnp.minimum(e, E - 1), 0)
    clamp3 = lambda e: (jnp.minimum(e, E - 1), 0, 0)
    return pl.pallas_call(
        _ffn_body,
        grid=(EB,),
        in_specs=[
            pl.BlockSpec((CAP, D), clamp),
            pl.BlockSpec((CAP, D), clamp),
            pl.BlockSpec((1, F, D), clamp3),
            pl.BlockSpec((1, D, F), clamp3),
            pl.BlockSpec((1, D, D), clamp3),
        ],
        out_specs=pl.BlockSpec((CAP, D), lambda e: (e, 0)),
        out_shape=jax.ShapeDtypeStruct((ROWS, D), jnp.float32),
    )(dk, dr, Wk, Wv, Wr)


def _combine_body(g_hbm, oe_hbm, out_hbm, g_v, rows_v, sem):
    w = lax.axis_index("s") * 2 + lax.axis_index("c")
    base_tok = w * TPT
    pltpu.sync_copy(g_hbm.at[pl.ds(base_tok, TPT)], g_v)
    pltpu.async_copy(oe_hbm.at[g_v], rows_v, sem).wait()
    pltpu.sync_copy(rows_v, out_hbm.at[pl.ds(base_tok, TPT)])


@functools.lru_cache
def _combine():
    mesh = plsc.VectorSubcoreMesh(core_axis_name="c", subcore_axis_name="s")
    return pl.kernel(
        _combine_body,
        out_type=jax.ShapeDtypeStruct((T, D), jnp.float32),
        mesh=mesh,
        scratch_types=[
            pltpu.VMEM((TPT,), jnp.int32),
            pltpu.VMEM((TPT, D), jnp.float32),
            pltpu.SemaphoreType.DMA,
        ],
    )


def kernel(x, token_ids, shift_state, time_maa_k, time_maa_r, Wk, Wv, Wr):
    xf = x.reshape(T, D)
    ss = shift_state.reshape(1, D)
    mk = time_maa_k.reshape(1, D)
    mr = time_maa_r.reshape(1, D)
    tid2d = token_ids.reshape(NTILES, TPT).astype(jnp.int32)
    xk, xr, g2d = _shift_and_route(xf, ss, mk, mr, tid2d)
    g = g2d.reshape(T)
    dk, dr = _dispatch()(g, xk, xr)
    oe = _ffn(dk, dr, Wk, Wv, Wr)
    out = _combine()(g, oe)
    return out.reshape(B, T, D), x[:, -1]


# speedup vs baseline: 2.8845x; 1.0558x over previous
import functools

import jax
import jax.numpy as jnp
from jax import lax
from jax.experimental import pallas as pl
from jax.experimental.pallas import tpu as pltpu
from jax.experimental.pallas import tpu_sc as plsc

B, T, D, F, E = 1, 2048, 768, 3072, 8
PRIME = 5099
CAP = T // E
NTILES = 32
TPT = T // NTILES
EB = E + 1
ROWS = EB * CAP
TRASH = E * CAP


def _route_body(tid_ref, g_ref, gn_ref, g0_ref):
    e2 = (tid_ref[...] * PRIME) & (E - 1)
    jr = lax.broadcasted_iota(jnp.int32, (TPT, TPT), 0)
    jc = lax.broadcasted_iota(jnp.int32, (TPT, TPT), 1)
    ltj = jnp.where(jr < jc, 1.0, 0.0)
    wr_ = lax.broadcasted_iota(jnp.int32, (NTILES, NTILES), 0)
    wc_ = lax.broadcasted_iota(jnp.int32, (NTILES, NTILES), 1)
    ltw = jnp.where(wr_ > wc_, 1.0, 0.0)
    onesr = jnp.full((TPT, NTILES), 1.0, jnp.float32)
    pos = jnp.zeros((NTILES, TPT), jnp.float32)
    for ei in range(E):
        m = jnp.where(e2 == ei, 1.0, 0.0)
        pre = lax.dot_general(m, ltj, (((1,), (0,)), ((), ())),
                              preferred_element_type=jnp.float32)
        rowtot = lax.dot_general(m, onesr, (((1,), (0,)), ((), ())),
                                 preferred_element_type=jnp.float32)
        rowbase = lax.dot_general(ltw, rowtot, (((1,), (0,)), ((), ())),
                                  preferred_element_type=jnp.float32)
        pos = pos + m * (pre + rowbase[:, 0:1])
    posi = pos.astype(jnp.int32)
    g = jnp.where(posi < CAP, e2 * CAP + posi, TRASH)
    g_ref[...] = g

    colio = lax.broadcasted_iota(jnp.int32, (NTILES, TPT), 1)
    rowio = lax.broadcasted_iota(jnp.int32, (NTILES, TPT), 0)
    gj = pltpu.roll(g, shift=TPT - 1, axis=1)
    nrf = pltpu.roll(g[:, 0:1], shift=NTILES - 1, axis=0)
    gn = jnp.where(colio == TPT - 1, nrf, gj)
    gn_ref[...] = jnp.where((colio == TPT - 1) & (rowio == NTILES - 1),
                            TRASH, gn)

    bi = lax.broadcasted_iota(jnp.int32, (1, 16), 1)
    g0_ref[...] = jnp.where(bi == 0, g[0:1, 0:1], TRASH)


def _route(tid2d):
    return pl.pallas_call(
        _route_body,
        out_shape=(
            jax.ShapeDtypeStruct((NTILES, TPT), jnp.int32),
            jax.ShapeDtypeStruct((NTILES, TPT), jnp.int32),
            jax.ShapeDtypeStruct((1, 16), jnp.int32),
        ),
    )(tid2d)


def _dispatch_body(g_hbm, gn_hbm, g0_hbm, x_hbm, ss_hbm, a_hbm, b_hbm,
                   g_v, gn_v, g0_v, xa_v, ss_v, sem_a, sem_b, sem_s):
    w = lax.axis_index("s") * 2 + lax.axis_index("c")
    base_tok = w * TPT
    pltpu.sync_copy(g_hbm.at[pl.ds(base_tok, TPT)], g_v)
    pltpu.sync_copy(gn_hbm.at[pl.ds(base_tok, TPT)], gn_v)
    pltpu.sync_copy(x_hbm.at[pl.ds(base_tok, TPT)], xa_v)
    ca = pltpu.async_copy(xa_v, a_hbm.at[g_v], sem_a)
    cb = pltpu.async_copy(xa_v, b_hbm.at[gn_v], sem_b)

    @pl.when(w == 0)
    def _():
        pltpu.sync_copy(g0_hbm, g0_v)
        pltpu.sync_copy(ss_hbm, ss_v.at[pl.ds(0, 1)])
        pltpu.async_copy(ss_v, b_hbm.at[g0_v], sem_s).wait()

    ca.wait()
    cb.wait()


@functools.lru_cache
def _dispatch():
    mesh = plsc.VectorSubcoreMesh(core_axis_name="c", subcore_axis_name="s")
    return pl.kernel(
        _dispatch_body,
        out_type=(
            jax.ShapeDtypeStruct((ROWS, D), jnp.float32),
            jax.ShapeDtypeStruct((ROWS, D), jnp.float32),
        ),
        mesh=mesh,
        scratch_types=[
            pltpu.VMEM((TPT,), jnp.int32),
            pltpu.VMEM((TPT,), jnp.int32),
            pltpu.VMEM((16,), jnp.int32),
            pltpu.VMEM((TPT, D), jnp.float32),
            pltpu.VMEM((16, D), jnp.float32),
            pltpu.SemaphoreType.DMA,
            pltpu.SemaphoreType.DMA,
            pltpu.SemaphoreType.DMA,
        ],
    )


def _ffn_body(a_ref, b_ref, mk_ref, mr_ref, wk_ref, wv_ref, wr_ref, oe_ref):
    e = pl.program_id(0)

    @pl.when(e < E)
    def _():
        a = a_ref[...]
        d_ = b_ref[...] - a
        dk = a + d_ * mk_ref[...]
        dr = a + d_ * mr_ref[...]
        h = lax.dot_general(dk, wk_ref[0], (((1,), (1,)), ((), ())),
                            preferred_element_type=jnp.float32)
        h = jnp.maximum(h, 0.0)
        h = h * h
        ev = lax.dot_general(h, wv_ref[0], (((1,), (1,)), ((), ())),
                             preferred_element_type=jnp.float32)
        er = lax.dot_general(dr, wr_ref[0], (((1,), (1,)), ((), ())),
                             preferred_element_type=jnp.float32)
        oe_ref[...] = ev * jax.nn.sigmoid(er)

    @pl.when(e == E)
    def _():
        oe_ref[...] = jnp.zeros_like(oe_ref)


def _ffn(a, b, mk, mr, Wk, Wv, Wr):
    clamp = lambda e: (jnp.minimum(e, E - 1), 0)
    clamp3 = lambda e: (jnp.minimum(e, E - 1), 0, 0)
    fixed = lambda e: (0, 0)
    return pl.pallas_call(
        _ffn_body,
        grid=(EB,),
        in_specs=[
            pl.BlockSpec((CAP, D), clamp),
            pl.BlockSpec((CAP, D), clamp),
            pl.BlockSpec((1, D), fixed),
            pl.BlockSpec((1, D), fixed),
            pl.BlockSpec((1, F, D), clamp3),
            pl.BlockSpec((1, D, F), clamp3),
            pl.BlockSpec((1, D, D), clamp3),
        ],
        out_specs=pl.BlockSpec((CAP, D), lambda e: (e, 0)),
        out_shape=jax.ShapeDtypeStruct((ROWS, D), jnp.float32),
    )(a, b, mk, mr, Wk, Wv, Wr)


def _combine_body(g_hbm, oe_hbm, out_hbm, g_v, rows_v, sem):
    w = lax.axis_index("s") * 2 + lax.axis_index("c")
    base_tok = w * TPT
    pltpu.sync_copy(g_hbm.at[pl.ds(base_tok, TPT)], g_v)
    pltpu.async_copy(oe_hbm.at[g_v], rows_v, sem).wait()
    pltpu.sync_copy(rows_v, out_hbm.at[pl.ds(base_tok, TPT)])


@functools.lru_cache
def _combine():
    mesh = plsc.VectorSubcoreMesh(core_axis_name="c", subcore_axis_name="s")
    return pl.kernel(
        _combine_body,
        out_type=jax.ShapeDtypeStruct((T, D), jnp.float32),
        mesh=mesh,
        scratch_types=[
            pltpu.VMEM((TPT,), jnp.int32),
            pltpu.VMEM((TPT, D), jnp.float32),
            pltpu.SemaphoreType.DMA,
        ],
    )


def kernel(x, token_ids, shift_state, time_maa_k, time_maa_r, Wk, Wv, Wr):
    xf = x.reshape(T, D)
    ss = shift_state.reshape(1, D)
    mk = time_maa_k.reshape(1, D)
    mr = time_maa_r.reshape(1, D)
    tid2d = token_ids.reshape(NTILES, TPT).astype(jnp.int32)
    g2d, gn2d, g0pad = _route(tid2d)
    g = g2d.reshape(T)
    a, b = _dispatch()(g, gn2d.reshape(T), g0pad.reshape(16), xf, ss)
    oe = _ffn(a, b, mk, mr, Wk, Wv, Wr)
    out = _combine()(g, oe)
    return out.reshape(B, T, D), x[:, -1]
